# baseline (speedup 1.0000x reference)
import functools

import jax
import jax.numpy as jnp
from jax import lax
from jax.experimental import pallas as pl
from jax.experimental.pallas import tpu as pltpu
from jax.experimental.pallas import tpu_sc as plsc

_L = 16


def _gather_call(B, V, D, NC, NS):
    NW = NC * NS
    n = B // NW
    G = n // _L
    mesh = plsc.VectorSubcoreMesh(core_axis_name="c", subcore_axis_name="s")

    @functools.partial(
        pl.kernel,
        mesh=mesh,
        out_type=jax.ShapeDtypeStruct((B, D), jnp.float32),
        scratch_types=[
            pltpu.VMEM((G, _L), jnp.int32),
            pltpu.VMEM((n, D), jnp.float32),
            [pltpu.SemaphoreType.DMA] * _L,
        ],
        compiler_params=pltpu.CompilerParams(needs_layout_passes=False),
    )
    def body(idx_hbm, table_hbm, out_hbm, idx_v, rows_v, sems):
        wid = lax.axis_index("s") * NC + lax.axis_index("c")
        base = wid * n
        pltpu.sync_copy(idx_hbm.at[wid], idx_v)
        iota = lax.iota(jnp.int32, _L)

        def group(g, _):
            vec = idx_v[g]
            for j in range(_L):
                i = jnp.sum(jnp.where(iota == j, vec, 0))
                pltpu.make_async_copy(
                    table_hbm.at[i], rows_v.at[g * _L + j], sems[j]
                ).start()
            return 0

        lax.fori_loop(0, G, group, 0)
        for j in range(_L):
            pltpu.make_async_copy(
                table_hbm.at[pl.ds(0, G)], rows_v.at[pl.ds(0, G)], sems[j]
            ).wait()
        pltpu.sync_copy(rows_v, out_hbm.at[pl.ds(base, n)])

    return body


def kernel(e, params):
    B = e.shape[0]
    V, D = params.shape
    info = plsc.get_sparse_core_info()
    NC, NS = info.num_cores, info.num_subcores
    NW = NC * NS
    idx = e.astype(jnp.int32).reshape(NW, (B // NW) // _L, _L)
    return _gather_call(B, V, D, NC, NS)(idx, params)


# speedup vs baseline: 1.4308x; 1.4308x over previous
import functools

import jax
import jax.numpy as jnp
from jax import lax
from jax.experimental import pallas as pl
from jax.experimental.pallas import tpu as pltpu
from jax.experimental.pallas import tpu_sc as plsc

_L = 16


def _gather_call(B, V, D, NC, NS):
    NW = NC * NS
    n = B // NW
    G = n // _L
    mesh = plsc.VectorSubcoreMesh(core_axis_name="c", subcore_axis_name="s")

    @functools.partial(
        pl.kernel,
        mesh=mesh,
        out_type=jax.ShapeDtypeStruct((B // _L, _L, D), jnp.float32),
        scratch_types=[
            pltpu.VMEM((G, _L), jnp.int32),
            pltpu.VMEM((G, _L, D), jnp.float32),
            [pltpu.SemaphoreType.DMA] * _L,
        ],
        compiler_params=pltpu.CompilerParams(needs_layout_passes=False),
    )
    def body(idx_hbm, table_hbm, out_hbm, idx_v, rows_v, sems):
        wid = lax.axis_index("s") * NC + lax.axis_index("c")
        base = wid * G
        pltpu.sync_copy(idx_hbm.at[wid], idx_v)
        iota = lax.iota(jnp.int32, _L)

        def group(g, _):
            vec = idx_v[g]
            for j in range(_L):
                i = jnp.sum(jnp.where(iota == j, vec, 0))
                pltpu.make_async_copy(
                    table_hbm.at[i >> 4, i & 15],
                    rows_v.at[g, j],
                    sems[j],
                ).start()
            return 0

        lax.fori_loop(0, G, group, 0)
        for j in range(_L):
            for q in range(G // _L):
                pltpu.make_async_copy(
                    table_hbm.at[0], rows_v.at[q], sems[j]
                ).wait()
        pltpu.sync_copy(rows_v, out_hbm.at[pl.ds(base, G)])

    return body


def kernel(e, params):
    B = e.shape[0]
    V, D = params.shape
    info = plsc.get_sparse_core_info()
    NC, NS = info.num_cores, info.num_subcores
    NW = NC * NS
    idx = e.astype(jnp.int32).reshape(NW, (B // NW) // _L, _L)
    table3 = params.reshape(V // _L, _L, D)
    out = _gather_call(B, V, D, NC, NS)(idx, table3)
    return out.reshape(B, D)


# speedup vs baseline: 1.6008x; 1.1188x over previous
import functools

import jax
import jax.numpy as jnp
from jax import lax
from jax.experimental import pallas as pl
from jax.experimental.pallas import tpu as pltpu
from jax.experimental.pallas import tpu_sc as plsc

_L = 16


def _gather_call(B, V, D, NC, NS):
    NW = NC * NS
    n = B // NW
    G = n // _L
    mesh = plsc.VectorSubcoreMesh(core_axis_name="c", subcore_axis_name="s")

    @functools.partial(
        pl.kernel,
        mesh=mesh,
        out_type=jax.ShapeDtypeStruct((B // _L, _L, D), jnp.float32),
        scratch_types=[
            pltpu.VMEM((G, _L), jnp.int32),
            pltpu.VMEM((G, _L, D), jnp.float32),
            pltpu.SemaphoreType.DMA,
        ],
        compiler_params=pltpu.CompilerParams(needs_layout_passes=False),
    )
    def body(idx_hbm, table_hbm, out_hbm, idx_v, rows_v, sem):
        wid = lax.axis_index("s") * NC + lax.axis_index("c")
        base = wid * G
        pltpu.sync_copy(idx_hbm.at[wid], idx_v)
        iota = lax.iota(jnp.int32, _L)

        def group(g, _):
            vec = idx_v[g]
            for j in range(_L):
                i = jnp.sum(jnp.where(iota == j, vec, 0))
                pltpu.make_async_copy(
                    table_hbm.at[i >> 4, i & 15],
                    rows_v.at[g, j],
                    sem,
                ).start()
            return 0

        lax.fori_loop(0, G, group, 0)
        pltpu.make_async_copy(
            table_hbm.at[pl.ds(0, G)], rows_v, sem
        ).wait()
        pltpu.sync_copy(rows_v, out_hbm.at[pl.ds(base, G)])

    return body


def kernel(e, params):
    B = e.shape[0]
    V, D = params.shape
    info = plsc.get_sparse_core_info()
    NC, NS = info.num_cores, info.num_subcores
    NW = NC * NS
    idx = e.astype(jnp.int32).reshape(NW, (B // NW) // _L, _L)
    table3 = params.reshape(V // _L, _L, D)
    out = _gather_call(B, V, D, NC, NS)(idx, table3)
    return out.reshape(B, D)


# speedup vs baseline: 1.6062x; 1.0034x over previous
import functools

import jax
import jax.numpy as jnp
from jax import lax
from jax.experimental import pallas as pl
from jax.experimental.pallas import tpu as pltpu
from jax.experimental.pallas import tpu_sc as plsc

_L = 16


def _gather_call(B, V, D, NC, NS):
    NW = NC * NS
    n = B // NW
    G = n // _L
    mesh = plsc.VectorSubcoreMesh(core_axis_name="c", subcore_axis_name="s")

    @functools.partial(
        pl.kernel,
        mesh=mesh,
        out_type=jax.ShapeDtypeStruct((B // _L, _L, D), jnp.float32),
        scratch_types=[
            pltpu.VMEM((G, _L), jnp.int32),
            pltpu.VMEM((G, _L, D), jnp.float32),
            [pltpu.SemaphoreType.DMA] * 2,
        ],
        compiler_params=pltpu.CompilerParams(needs_layout_passes=False),
    )
    def body(idx_hbm, table_hbm, out_hbm, idx_v, rows_v, sems):
        wid = lax.axis_index("s") * NC + lax.axis_index("c")
        base = wid * G
        pltpu.sync_copy(idx_hbm.at[wid], idx_v)
        iota = lax.iota(jnp.int32, _L)
        H = G // 2

        def group(g, _, sem=None):
            vec = idx_v[g]
            for j in range(_L):
                i = jnp.sum(jnp.where(iota == j, vec, 0))
                pltpu.make_async_copy(
                    table_hbm.at[i >> 4, i & 15],
                    rows_v.at[g, j],
                    sem,
                ).start()
            return 0

        lax.fori_loop(0, H, functools.partial(group, sem=sems[0]), 0)
        lax.fori_loop(H, G, functools.partial(group, sem=sems[1]), 0)
        pltpu.make_async_copy(
            table_hbm.at[pl.ds(0, H)], rows_v.at[pl.ds(0, H)], sems[0]
        ).wait()
        pltpu.sync_copy(rows_v.at[pl.ds(0, H)], out_hbm.at[pl.ds(base, H)])
        pltpu.make_async_copy(
            table_hbm.at[pl.ds(0, H)], rows_v.at[pl.ds(H, H)], sems[1]
        ).wait()
        pltpu.sync_copy(rows_v.at[pl.ds(H, H)], out_hbm.at[pl.ds(base + H, H)])

    return body


def kernel(e, params):
    B = e.shape[0]
    V, D = params.shape
    info = plsc.get_sparse_core_info()
    NC, NS = info.num_cores, info.num_subcores
    NW = NC * NS
    idx = e.astype(jnp.int32).reshape(NW, (B // NW) // _L, _L)
    table3 = params.reshape(V // _L, _L, D)
    out = _gather_call(B, V, D, NC, NS)(idx, table3)
    return out.reshape(B, D)
